# Initial kernel scaffold; baseline (speedup 1.0000x reference)
import jax
import jax.numpy as jnp
from jax.experimental import pallas as pl


def kernel(embeddings, edge_index, Wq, bq, Wk, bk, Wv, bv):
    raise NotImplementedError("write your pallas kernel here")


# speedup vs baseline: 1.1374x; 1.1374x over previous
import jax
import jax.numpy as jnp
from jax import lax
from jax.experimental import pallas as pl
from jax.experimental.pallas import tpu as pltpu
from jax.experimental.pallas import tpu_sc as plsc

N_NODES = 10000
N_EDGES = 160000
D = 256
NC, NS, L = 2, 16, 16
NW = NC * NS
NP = 10240
NE_PAD = 163840

_SC_PARAMS = pltpu.CompilerParams(needs_layout_passes=False)


def _sc_mesh():
    return plsc.VectorSubcoreMesh(core_axis_name="c", subcore_axis_name="s",
                                  num_cores=NC, num_subcores=NS)


def _proj_body(x_ref, w_ref, b_ref, wv_ref, bv_ref, q_ref, k_ref, vt_ref):
    x = x_ref[...]
    y = jnp.dot(x, w_ref[...], preferred_element_type=jnp.float32) + b_ref[...]
    q_ref[...] = y[:, :D]
    k_ref[...] = y[:, D:]
    vt = lax.dot_general(wv_ref[...], x, (((1,), (1,)), ((), ())),
                         preferred_element_type=jnp.float32)
    vt_ref[...] = vt + bv_ref[...]


def _project(emb_pad, wqk, bqk, wv, bvcol):
    R = 512
    return pl.pallas_call(
        _proj_body,
        grid=(NP // R,),
        in_specs=[
            pl.BlockSpec((R, D), lambda i: (i, 0)),
            pl.BlockSpec((D, 2 * D), lambda i: (0, 0)),
            pl.BlockSpec((1, 2 * D), lambda i: (0, 0)),
            pl.BlockSpec((D, D), lambda i: (0, 0)),
            pl.BlockSpec((D, 1), lambda i: (0, 0)),
        ],
        out_specs=[
            pl.BlockSpec((R, D), lambda i: (i, 0)),
            pl.BlockSpec((R, D), lambda i: (i, 0)),
            pl.BlockSpec((D, R), lambda i: (0, i)),
        ],
        out_shape=[
            jax.ShapeDtypeStruct((NP, D), jnp.float32),
            jax.ShapeDtypeStruct((NP, D), jnp.float32),
            jax.ShapeDtypeStruct((D, NP), jnp.float32),
        ],
    )(emb_pad, wqk, bqk, wv, bvcol)


EPW = NE_PAD // NW
C2 = 64
NCH2 = EPW // C2


def _scores_body(q_hbm, k_hbm, dst_hbm, src_hbm, s_hbm,
                 di_v, si_v, qrows, krows, sv, pacc, sem):
    cid = lax.axis_index("c")
    sid = lax.axis_index("s")
    base = (sid * NC + cid) * EPW

    def chunk(i, carry):
        off = base + i * C2
        pltpu.sync_copy(dst_hbm.at[pl.ds(off, C2)], di_v)
        pltpu.sync_copy(src_hbm.at[pl.ds(off, C2)], si_v)
        cp_q = pltpu.async_copy(q_hbm.at[di_v], qrows, sem)
        cp_k = pltpu.async_copy(k_hbm.at[si_v], krows, sem)
        cp_q.wait()
        cp_k.wait()
        lane = lax.iota(jnp.int32, L)

        def group(g, c):
            for e16 in range(L):
                e = g * L + e16
                acc = qrows[e, pl.ds(0, L)] * krows[e, pl.ds(0, L)]
                for j in range(1, D // L):
                    acc = acc + qrows[e, pl.ds(j * L, L)] * krows[e, pl.ds(j * L, L)]
                pacc[e16, :] = acc
            svec = plsc.load_gather(pacc, [lane, jnp.zeros((L,), jnp.int32)])
            for j in range(1, L):
                svec = svec + plsc.load_gather(pacc, [lane, jnp.full((L,), j, jnp.int32)])
            sv[pl.ds(g * L, L)] = svec
            return c

        lax.fori_loop(0, C2 // L, group, 0, unroll=False)
        pltpu.sync_copy(sv, s_hbm.at[pl.ds(off, C2)])
        return carry

    lax.fori_loop(0, NCH2, chunk, 0, unroll=False)


def _scores(qs, ks, dst, src):
    return pl.kernel(
        _scores_body,
        out_type=jax.ShapeDtypeStruct((NE_PAD,), jnp.float32),
        mesh=_sc_mesh(),
        compiler_params=_SC_PARAMS,
        scratch_types=[
            pltpu.VMEM((C2,), jnp.int32),
            pltpu.VMEM((C2,), jnp.int32),
            pltpu.VMEM((C2, D), jnp.float32),
            pltpu.VMEM((C2, D), jnp.float32),
            pltpu.VMEM((C2,), jnp.float32),
            pltpu.VMEM((L, L), jnp.float32),
            pltpu.SemaphoreType.DMA,
        ],
    )(qs, ks, dst, src)


SM_ROWS = NE_PAD // 128


def _softmax_body(s_ref, w_ref):
    s = s_ref[...]
    rows = lax.broadcasted_iota(jnp.int32, (SM_ROWS, 128), 0)
    cols = lax.broadcasted_iota(jnp.int32, (SM_ROWS, 128), 1)
    valid = rows * 128 + cols < N_EDGES
    s = jnp.where(valid, s, -jnp.inf)
    m = jnp.max(s)
    e = jnp.where(valid, jnp.exp(s - m), 0.0)
    w_ref[...] = e / jnp.sum(e)


def _softmax(scores):
    return pl.pallas_call(
        _softmax_body,
        out_shape=jax.ShapeDtypeStruct((SM_ROWS, 128), jnp.float32),
    )(scores.reshape(SM_ROWS, 128))


SLAB = 4
NPASS = D // (SLAB * NW)
CE = 2000
NCH4 = N_EDGES // CE


def _agg_body(vt_hbm, src_hbm, dst_hbm, w_hbm, agg_hbm,
              slab, acc, si_v, di_v, wv, sem):
    cid = lax.axis_index("c")
    sid = lax.axis_index("s")
    wid = sid * NC + cid
    zero = jnp.zeros((L,), jnp.float32)
    for p in range(NPASS):
        slab_id = p * NW + wid
        pltpu.sync_copy(vt_hbm.at[slab_id], slab)

        def zinit(i, c):
            for j in range(SLAB):
                acc[pl.ds(j * NP + i * L, L)] = zero
            return c

        lax.fori_loop(0, NP // L, zinit, 0, unroll=False)

        def chunk(i, c):
            off = i * CE
            pltpu.sync_copy(src_hbm.at[pl.ds(off, CE)], si_v)
            pltpu.sync_copy(dst_hbm.at[pl.ds(off, CE)], di_v)
            pltpu.sync_copy(w_hbm.at[pl.ds(off, CE)], wv)

            def grp(g, c2):
                s16 = si_v[pl.ds(g * L, L)]
                d16 = di_v[pl.ds(g * L, L)]
                w16 = wv[pl.ds(g * L, L)]
                for j in range(SLAB):
                    v = plsc.load_gather(slab, [s16 + (j * NP)])
                    plsc.addupdate_scatter(acc, [d16 + (j * NP)], v * w16)
                return c2

            lax.fori_loop(0, CE // L, grp, 0, unroll=False)
            return c

        lax.fori_loop(0, NCH4, chunk, 0, unroll=False)
        pltpu.sync_copy(acc, agg_hbm.at[slab_id])


def _aggregate(vt_slabs, src, dst, w):
    return pl.kernel(
        _agg_body,
        out_type=jax.ShapeDtypeStruct((NPASS * NW, SLAB * NP), jnp.float32),
        mesh=_sc_mesh(),
        compiler_params=_SC_PARAMS,
        scratch_types=[
            pltpu.VMEM((SLAB * NP,), jnp.float32),
            pltpu.VMEM((SLAB * NP,), jnp.float32),
            pltpu.VMEM((CE,), jnp.int32),
            pltpu.VMEM((CE,), jnp.int32),
            pltpu.VMEM((CE,), jnp.float32),
            pltpu.SemaphoreType.DMA,
        ],
    )(vt_slabs, src, dst, w)


def _final_body(agg_ref, emb_ref, out_ref):
    out_ref[...] = emb_ref[...] + lax.transpose(agg_ref[...], (1, 0))


def _finalize(agg_t, emb_pad):
    R = 512
    return pl.pallas_call(
        _final_body,
        grid=(NP // R,),
        in_specs=[
            pl.BlockSpec((D, R), lambda i: (0, i)),
            pl.BlockSpec((R, D), lambda i: (i, 0)),
        ],
        out_specs=pl.BlockSpec((R, D), lambda i: (i, 0)),
        out_shape=jax.ShapeDtypeStruct((NP, D), jnp.float32),
    )(agg_t, emb_pad)


def kernel(embeddings, edge_index, Wq, bq, Wk, bk, Wv, bv):
    inv = 1.0 / (D ** 0.5)
    wqk = jnp.concatenate([Wq.T * inv, Wk.T], axis=1)
    bqk = jnp.concatenate([bq * inv, bk]).reshape(1, 2 * D)
    bvcol = bv.reshape(D, 1)

    src = edge_index[0].astype(jnp.int32)
    dst = edge_index[1].astype(jnp.int32)
    src_p = jnp.pad(src, (0, NE_PAD - N_EDGES))
    dst_p = jnp.pad(dst, (0, NE_PAD - N_EDGES))
    emb_pad = jnp.pad(embeddings, ((0, NP - N_NODES), (0, 0)))

    qs, ks, vt = _project(emb_pad, wqk, bqk, Wv, bvcol)
    scores = _scores(qs, ks, dst_p, src_p)
    w = _softmax(scores).reshape(NE_PAD)

    vt_slabs = vt.reshape(NPASS * NW, SLAB * NP)
    agg = _aggregate(vt_slabs, src, dst, w[:N_EDGES])
    out = _finalize(agg.reshape(D, NP), emb_pad)
    return out[:N_NODES]


# speedup vs baseline: 1.4142x; 1.2434x over previous
import jax
import jax.numpy as jnp
from jax import lax
from jax.experimental import pallas as pl
from jax.experimental.pallas import tpu as pltpu
from jax.experimental.pallas import tpu_sc as plsc

N_NODES = 10000
N_EDGES = 160000
D = 256
NC, NS, L = 2, 16, 16
NW = NC * NS
NP = 10240
NE_PAD = 163840

_SC_PARAMS = pltpu.CompilerParams(needs_layout_passes=False)


def _sc_mesh():
    return plsc.VectorSubcoreMesh(core_axis_name="c", subcore_axis_name="s",
                                  num_cores=NC, num_subcores=NS)


def _proj_body(x_ref, w_ref, b_ref, wv_ref, bv_ref, q_ref, k_ref, vt_ref):
    x = x_ref[...]
    y = jnp.dot(x, w_ref[...], preferred_element_type=jnp.float32) + b_ref[...]
    q_ref[...] = y[:, :D]
    k_ref[...] = y[:, D:]
    vt = lax.dot_general(wv_ref[...], x, (((1,), (1,)), ((), ())),
                         preferred_element_type=jnp.float32)
    vt_ref[...] = vt + bv_ref[...]


def _project(emb_pad, wqk, bqk, wv, bvcol):
    R = 512
    return pl.pallas_call(
        _proj_body,
        grid=(NP // R,),
        in_specs=[
            pl.BlockSpec((R, D), lambda i: (i, 0)),
            pl.BlockSpec((D, 2 * D), lambda i: (0, 0)),
            pl.BlockSpec((1, 2 * D), lambda i: (0, 0)),
            pl.BlockSpec((D, D), lambda i: (0, 0)),
            pl.BlockSpec((D, 1), lambda i: (0, 0)),
        ],
        out_specs=[
            pl.BlockSpec((R, D), lambda i: (i, 0)),
            pl.BlockSpec((R, D), lambda i: (i, 0)),
            pl.BlockSpec((D, R), lambda i: (0, i)),
        ],
        out_shape=[
            jax.ShapeDtypeStruct((NP, D), jnp.float32),
            jax.ShapeDtypeStruct((NP, D), jnp.float32),
            jax.ShapeDtypeStruct((D, NP), jnp.float32),
        ],
    )(emb_pad, wqk, bqk, wv, bvcol)


EPW = NE_PAD // NW
C2 = 64
NCH2 = EPW // C2


def _scores_body(q_hbm, k_hbm, dst_hbm, src_hbm, s_hbm,
                 di_v, si_v, qrows, krows, sv, pacc, sem):
    cid = lax.axis_index("c")
    sid = lax.axis_index("s")
    base = (sid * NC + cid) * EPW

    def chunk(i, carry):
        off = base + i * C2
        pltpu.sync_copy(dst_hbm.at[pl.ds(off, C2)], di_v)
        pltpu.sync_copy(src_hbm.at[pl.ds(off, C2)], si_v)
        cp_q = pltpu.async_copy(q_hbm.at[di_v], qrows, sem)
        cp_k = pltpu.async_copy(k_hbm.at[si_v], krows, sem)
        cp_q.wait()
        cp_k.wait()
        lane = lax.iota(jnp.int32, L)

        def group(g, c):
            for e16 in range(L):
                e = g * L + e16
                acc = qrows[e, pl.ds(0, L)] * krows[e, pl.ds(0, L)]
                for j in range(1, D // L):
                    acc = acc + qrows[e, pl.ds(j * L, L)] * krows[e, pl.ds(j * L, L)]
                pacc[e16, :] = acc
            svec = plsc.load_gather(pacc, [lane, jnp.zeros((L,), jnp.int32)])
            for j in range(1, L):
                svec = svec + plsc.load_gather(pacc, [lane, jnp.full((L,), j, jnp.int32)])
            sv[pl.ds(g * L, L)] = svec
            return c

        lax.fori_loop(0, C2 // L, group, 0, unroll=False)
        pltpu.sync_copy(sv, s_hbm.at[pl.ds(off, C2)])
        return carry

    lax.fori_loop(0, NCH2, chunk, 0, unroll=False)


def _scores(qs, ks, dst, src):
    return pl.kernel(
        _scores_body,
        out_type=jax.ShapeDtypeStruct((NE_PAD,), jnp.float32),
        mesh=_sc_mesh(),
        compiler_params=_SC_PARAMS,
        scratch_types=[
            pltpu.VMEM((C2,), jnp.int32),
            pltpu.VMEM((C2,), jnp.int32),
            pltpu.VMEM((C2, D), jnp.float32),
            pltpu.VMEM((C2, D), jnp.float32),
            pltpu.VMEM((C2,), jnp.float32),
            pltpu.VMEM((L, L), jnp.float32),
            pltpu.SemaphoreType.DMA,
        ],
    )(qs, ks, dst, src)


SM_ROWS = NE_PAD // 128


def _softmax_body(s_ref, w_ref):
    s = s_ref[...]
    rows = lax.broadcasted_iota(jnp.int32, (SM_ROWS, 128), 0)
    cols = lax.broadcasted_iota(jnp.int32, (SM_ROWS, 128), 1)
    valid = rows * 128 + cols < N_EDGES
    s = jnp.where(valid, s, -jnp.inf)
    m = jnp.max(s)
    e = jnp.where(valid, jnp.exp(s - m), 0.0)
    w_ref[...] = e / jnp.sum(e)


def _softmax(scores):
    return pl.pallas_call(
        _softmax_body,
        out_shape=jax.ShapeDtypeStruct((SM_ROWS, 128), jnp.float32),
    )(scores.reshape(SM_ROWS, 128))


SLAB = 4
NPASS = D // (SLAB * NW)
CE = 2000
NCH4 = N_EDGES // CE


GU = 5


def _agg_body(vt_hbm, src_hbm, dst_hbm, w_hbm, agg_hbm,
              slab, acc, si0, di0, wv0, si1, di1, wv1, sem0, sem1, sems):
    cid = lax.axis_index("c")
    sid = lax.axis_index("s")
    wid = sid * NC + cid
    zero = jnp.zeros((L,), jnp.float32)
    bufs = ((si0, di0, wv0, sem0), (si1, di1, wv1, sem1))

    def issue(i, b):
        si, di, wv, sem = bufs[b]
        off = i * CE
        pltpu.async_copy(src_hbm.at[pl.ds(off, CE)], si, sem)
        pltpu.async_copy(dst_hbm.at[pl.ds(off, CE)], di, sem)
        pltpu.async_copy(w_hbm.at[pl.ds(off, CE)], wv, sem)

    def wait(b):
        si, di, wv, sem = bufs[b]
        pltpu.make_async_copy(src_hbm.at[pl.ds(0, CE)], si, sem).wait()
        pltpu.make_async_copy(dst_hbm.at[pl.ds(0, CE)], di, sem).wait()
        pltpu.make_async_copy(w_hbm.at[pl.ds(0, CE)], wv, sem).wait()

    def compute(b):
        si, di, wv, _ = bufs[b]

        def grp(i, c2):
            for u in range(GU):
                g = i * GU + u
                s16 = si[pl.ds(g * L, L)]
                d16 = di[pl.ds(g * L, L)]
                w16 = wv[pl.ds(g * L, L)]
                for j in range(SLAB):
                    v = plsc.load_gather(slab, [s16 + (j * NP)])
                    plsc.addupdate_scatter(acc, [d16 + (j * NP)], v * w16)
            return c2

        lax.fori_loop(0, CE // (L * GU), grp, 0, unroll=False)

    for p in range(NPASS):
        slab_id = p * NW + wid
        cp_slab = pltpu.async_copy(vt_hbm.at[slab_id], slab, sems)
        issue(0, 0)

        def zinit(i, c):
            for j in range(SLAB):
                acc[pl.ds(j * NP + i * L, L)] = zero
            return c

        lax.fori_loop(0, NP // L, zinit, 0, unroll=False)
        cp_slab.wait()

        def outer(i2, c):
            ia = 2 * i2
            issue(ia + 1, 1)
            wait(0)
            compute(0)

            @pl.when(ia + 2 < NCH4)
            def _():
                issue(ia + 2, 0)

            wait(1)
            compute(1)
            return c

        lax.fori_loop(0, NCH4 // 2, outer, 0, unroll=False)
        pltpu.sync_copy(acc, agg_hbm.at[slab_id])


def _aggregate(vt_slabs, src, dst, w):
    return pl.kernel(
        _agg_body,
        out_type=jax.ShapeDtypeStruct((NPASS * NW, SLAB * NP), jnp.float32),
        mesh=_sc_mesh(),
        compiler_params=_SC_PARAMS,
        scratch_types=[
            pltpu.VMEM((SLAB * NP,), jnp.float32),
            pltpu.VMEM((SLAB * NP,), jnp.float32),
            pltpu.VMEM((CE,), jnp.int32),
            pltpu.VMEM((CE,), jnp.int32),
            pltpu.VMEM((CE,), jnp.float32),
            pltpu.VMEM((CE,), jnp.int32),
            pltpu.VMEM((CE,), jnp.int32),
            pltpu.VMEM((CE,), jnp.float32),
            pltpu.SemaphoreType.DMA,
            pltpu.SemaphoreType.DMA,
            pltpu.SemaphoreType.DMA,
        ],
    )(vt_slabs, src, dst, w)


def _final_body(agg_ref, emb_ref, out_ref):
    out_ref[...] = emb_ref[...] + lax.transpose(agg_ref[...], (1, 0))


def _finalize(agg_t, emb_pad):
    R = 512
    return pl.pallas_call(
        _final_body,
        grid=(NP // R,),
        in_specs=[
            pl.BlockSpec((D, R), lambda i: (0, i)),
            pl.BlockSpec((R, D), lambda i: (i, 0)),
        ],
        out_specs=pl.BlockSpec((R, D), lambda i: (i, 0)),
        out_shape=jax.ShapeDtypeStruct((NP, D), jnp.float32),
    )(agg_t, emb_pad)


def kernel(embeddings, edge_index, Wq, bq, Wk, bk, Wv, bv):
    inv = 1.0 / (D ** 0.5)
    wqk = jnp.concatenate([Wq.T * inv, Wk.T], axis=1)
    bqk = jnp.concatenate([bq * inv, bk]).reshape(1, 2 * D)
    bvcol = bv.reshape(D, 1)

    src = edge_index[0].astype(jnp.int32)
    dst = edge_index[1].astype(jnp.int32)
    src_p = jnp.pad(src, (0, NE_PAD - N_EDGES))
    dst_p = jnp.pad(dst, (0, NE_PAD - N_EDGES))
    emb_pad = jnp.pad(embeddings, ((0, NP - N_NODES), (0, 0)))

    qs, ks, vt = _project(emb_pad, wqk, bqk, Wv, bvcol)
    scores = _scores(qs, ks, dst_p, src_p)
    w = _softmax(scores).reshape(NE_PAD)

    vt_slabs = vt.reshape(NPASS * NW, SLAB * NP)
    agg = _aggregate(vt_slabs, src, dst, w[:N_EDGES])
    out = _finalize(agg.reshape(D, NP), emb_pad)
    return out[:N_NODES]


# speedup vs baseline: 1.4159x; 1.0012x over previous
import jax
import jax.numpy as jnp
from jax import lax
from jax.experimental import pallas as pl
from jax.experimental.pallas import tpu as pltpu
from jax.experimental.pallas import tpu_sc as plsc

N_NODES = 10000
N_EDGES = 160000
D = 256
NC, NS, L = 2, 16, 16
NW = NC * NS
NP = 10240
NE_PAD = 163840

_SC_PARAMS = pltpu.CompilerParams(needs_layout_passes=False)


def _sc_mesh():
    return plsc.VectorSubcoreMesh(core_axis_name="c", subcore_axis_name="s",
                                  num_cores=NC, num_subcores=NS)


def _proj_body(x_ref, w_ref, b_ref, wv_ref, bv_ref, q_ref, k_ref, vt_ref):
    x = x_ref[...]
    y = jnp.dot(x, w_ref[...], preferred_element_type=jnp.float32) + b_ref[...]
    q_ref[...] = y[:, :D]
    k_ref[...] = y[:, D:]
    vt = lax.dot_general(wv_ref[...], x, (((1,), (1,)), ((), ())),
                         preferred_element_type=jnp.float32)
    vt_ref[...] = vt + bv_ref[...]


def _project(emb_pad, wqk, bqk, wv, bvcol):
    R = 512
    return pl.pallas_call(
        _proj_body,
        grid=(NP // R,),
        in_specs=[
            pl.BlockSpec((R, D), lambda i: (i, 0)),
            pl.BlockSpec((D, 2 * D), lambda i: (0, 0)),
            pl.BlockSpec((1, 2 * D), lambda i: (0, 0)),
            pl.BlockSpec((D, D), lambda i: (0, 0)),
            pl.BlockSpec((D, 1), lambda i: (0, 0)),
        ],
        out_specs=[
            pl.BlockSpec((R, D), lambda i: (i, 0)),
            pl.BlockSpec((R, D), lambda i: (i, 0)),
            pl.BlockSpec((D, R), lambda i: (0, i)),
        ],
        out_shape=[
            jax.ShapeDtypeStruct((NP, D), jnp.float32),
            jax.ShapeDtypeStruct((NP, D), jnp.float32),
            jax.ShapeDtypeStruct((D, NP), jnp.float32),
        ],
    )(emb_pad, wqk, bqk, wv, bvcol)


EPW = NE_PAD // NW
C2 = 64
NCH2 = EPW // C2


def _scores_body(q_hbm, k_hbm, dst_hbm, src_hbm, s_hbm,
                 di_v, si_v, qrows, krows, sv, pacc, sem):
    cid = lax.axis_index("c")
    sid = lax.axis_index("s")
    base = (sid * NC + cid) * EPW

    def chunk(i, carry):
        off = base + i * C2
        pltpu.sync_copy(dst_hbm.at[pl.ds(off, C2)], di_v)
        pltpu.sync_copy(src_hbm.at[pl.ds(off, C2)], si_v)
        cp_q = pltpu.async_copy(q_hbm.at[di_v], qrows, sem)
        cp_k = pltpu.async_copy(k_hbm.at[si_v], krows, sem)
        cp_q.wait()
        cp_k.wait()
        lane = lax.iota(jnp.int32, L)

        def group(g, c):
            for e16 in range(L):
                e = g * L + e16
                acc = qrows[e, pl.ds(0, L)] * krows[e, pl.ds(0, L)]
                for j in range(1, D // L):
                    acc = acc + qrows[e, pl.ds(j * L, L)] * krows[e, pl.ds(j * L, L)]
                pacc[e16, :] = acc
            svec = plsc.load_gather(pacc, [lane, jnp.zeros((L,), jnp.int32)])
            for j in range(1, L):
                svec = svec + plsc.load_gather(pacc, [lane, jnp.full((L,), j, jnp.int32)])
            sv[pl.ds(g * L, L)] = svec
            return c

        lax.fori_loop(0, C2 // L, group, 0, unroll=False)
        pltpu.sync_copy(sv, s_hbm.at[pl.ds(off, C2)])
        return carry

    lax.fori_loop(0, NCH2, chunk, 0, unroll=False)


def _scores(qs, ks, dst, src):
    return pl.kernel(
        _scores_body,
        out_type=jax.ShapeDtypeStruct((NE_PAD,), jnp.float32),
        mesh=_sc_mesh(),
        compiler_params=_SC_PARAMS,
        scratch_types=[
            pltpu.VMEM((C2,), jnp.int32),
            pltpu.VMEM((C2,), jnp.int32),
            pltpu.VMEM((C2, D), jnp.float32),
            pltpu.VMEM((C2, D), jnp.float32),
            pltpu.VMEM((C2,), jnp.float32),
            pltpu.VMEM((L, L), jnp.float32),
            pltpu.SemaphoreType.DMA,
        ],
    )(qs, ks, dst, src)


SM_ROWS = NE_PAD // 128


def _softmax_body(s_ref, w_ref):
    s = s_ref[...]
    rows = lax.broadcasted_iota(jnp.int32, (SM_ROWS, 128), 0)
    cols = lax.broadcasted_iota(jnp.int32, (SM_ROWS, 128), 1)
    valid = rows * 128 + cols < N_EDGES
    s = jnp.where(valid, s, -jnp.inf)
    m = jnp.max(s)
    e = jnp.where(valid, jnp.exp(s - m), 0.0)
    w_ref[...] = e / jnp.sum(e)


def _softmax(scores):
    return pl.pallas_call(
        _softmax_body,
        out_shape=jax.ShapeDtypeStruct((SM_ROWS, 128), jnp.float32),
    )(scores.reshape(SM_ROWS, 128))


SLAB = 4
NPASS = D // (SLAB * NW)
CE = 2000
NCH4 = N_EDGES // CE


GU = 25


def _agg_body(vt_hbm, src_hbm, dst_hbm, w_hbm, agg_hbm,
              slab, acc, si0, di0, wv0, si1, di1, wv1, sem0, sem1, sems):
    cid = lax.axis_index("c")
    sid = lax.axis_index("s")
    wid = sid * NC + cid
    zero = jnp.zeros((L,), jnp.float32)
    bufs = ((si0, di0, wv0, sem0), (si1, di1, wv1, sem1))

    def issue(i, b):
        si, di, wv, sem = bufs[b]
        off = i * CE
        pltpu.async_copy(src_hbm.at[pl.ds(off, CE)], si, sem)
        pltpu.async_copy(dst_hbm.at[pl.ds(off, CE)], di, sem)
        pltpu.async_copy(w_hbm.at[pl.ds(off, CE)], wv, sem)

    def wait(b):
        si, di, wv, sem = bufs[b]
        pltpu.make_async_copy(src_hbm.at[pl.ds(0, CE)], si, sem).wait()
        pltpu.make_async_copy(dst_hbm.at[pl.ds(0, CE)], di, sem).wait()
        pltpu.make_async_copy(w_hbm.at[pl.ds(0, CE)], wv, sem).wait()

    def compute(b):
        si, di, wv, _ = bufs[b]

        def grp(i, c2):
            for u in range(GU):
                g = i * GU + u
                s16 = si[pl.ds(g * L, L)]
                d16 = di[pl.ds(g * L, L)]
                w16 = wv[pl.ds(g * L, L)]
                for j in range(SLAB):
                    v = plsc.load_gather(slab, [s16 + (j * NP)])
                    plsc.addupdate_scatter(acc, [d16 + (j * NP)], v * w16)
            return c2

        lax.fori_loop(0, CE // (L * GU), grp, 0, unroll=False)

    for p in range(NPASS):
        slab_id = p * NW + wid
        cp_slab = pltpu.async_copy(vt_hbm.at[slab_id], slab, sems)
        issue(0, 0)

        def zinit(i, c):
            for j in range(SLAB):
                acc[pl.ds(j * NP + i * L, L)] = zero
            return c

        lax.fori_loop(0, NP // L, zinit, 0, unroll=False)
        cp_slab.wait()

        def outer(i2, c):
            ia = 2 * i2
            issue(ia + 1, 1)
            wait(0)
            compute(0)

            @pl.when(ia + 2 < NCH4)
            def _():
                issue(ia + 2, 0)

            wait(1)
            compute(1)
            return c

        lax.fori_loop(0, NCH4 // 2, outer, 0, unroll=False)
        pltpu.sync_copy(acc, agg_hbm.at[slab_id])


def _aggregate(vt_slabs, src, dst, w):
    return pl.kernel(
        _agg_body,
        out_type=jax.ShapeDtypeStruct((NPASS * NW, SLAB * NP), jnp.float32),
        mesh=_sc_mesh(),
        compiler_params=_SC_PARAMS,
        scratch_types=[
            pltpu.VMEM((SLAB * NP,), jnp.float32),
            pltpu.VMEM((SLAB * NP,), jnp.float32),
            pltpu.VMEM((CE,), jnp.int32),
            pltpu.VMEM((CE,), jnp.int32),
            pltpu.VMEM((CE,), jnp.float32),
            pltpu.VMEM((CE,), jnp.int32),
            pltpu.VMEM((CE,), jnp.int32),
            pltpu.VMEM((CE,), jnp.float32),
            pltpu.SemaphoreType.DMA,
            pltpu.SemaphoreType.DMA,
            pltpu.SemaphoreType.DMA,
        ],
    )(vt_slabs, src, dst, w)


def _final_body(agg_ref, emb_ref, out_ref):
    out_ref[...] = emb_ref[...] + lax.transpose(agg_ref[...], (1, 0))


def _finalize(agg_t, emb_pad):
    R = 512
    return pl.pallas_call(
        _final_body,
        grid=(NP // R,),
        in_specs=[
            pl.BlockSpec((D, R), lambda i: (0, i)),
            pl.BlockSpec((R, D), lambda i: (i, 0)),
        ],
        out_specs=pl.BlockSpec((R, D), lambda i: (i, 0)),
        out_shape=jax.ShapeDtypeStruct((NP, D), jnp.float32),
    )(agg_t, emb_pad)


def kernel(embeddings, edge_index, Wq, bq, Wk, bk, Wv, bv):
    inv = 1.0 / (D ** 0.5)
    wqk = jnp.concatenate([Wq.T * inv, Wk.T], axis=1)
    bqk = jnp.concatenate([bq * inv, bk]).reshape(1, 2 * D)
    bvcol = bv.reshape(D, 1)

    src = edge_index[0].astype(jnp.int32)
    dst = edge_index[1].astype(jnp.int32)
    src_p = jnp.pad(src, (0, NE_PAD - N_EDGES))
    dst_p = jnp.pad(dst, (0, NE_PAD - N_EDGES))
    emb_pad = jnp.pad(embeddings, ((0, NP - N_NODES), (0, 0)))

    qs, ks, vt = _project(emb_pad, wqk, bqk, Wv, bvcol)
    scores = _scores(qs, ks, dst_p, src_p)
    w = _softmax(scores).reshape(NE_PAD)

    vt_slabs = vt.reshape(NPASS * NW, SLAB * NP)
    agg = _aggregate(vt_slabs, src, dst, w[:N_EDGES])
    out = _finalize(agg.reshape(D, NP), emb_pad)
    return out[:N_NODES]


# speedup vs baseline: 1.5804x; 1.1161x over previous
import jax
import jax.numpy as jnp
from jax import lax
from jax.experimental import pallas as pl
from jax.experimental.pallas import tpu as pltpu
from jax.experimental.pallas import tpu_sc as plsc

N_NODES = 10000
N_EDGES = 160000
D = 256
NC, NS, L = 2, 16, 16
NW = NC * NS
NP = 10240
NE_PAD = 163840

_SC_PARAMS = pltpu.CompilerParams(needs_layout_passes=False)


def _sc_mesh():
    return plsc.VectorSubcoreMesh(core_axis_name="c", subcore_axis_name="s",
                                  num_cores=NC, num_subcores=NS)


def _bf16_bits(x):
    return lax.bitcast_convert_type(x.astype(jnp.bfloat16).astype(jnp.float32),
                                    jnp.int32)


def _proj_body(x_ref, w_ref, b_ref, wve_ref, wvo_ref, bve_ref, bvo_ref,
               q_ref, k_ref, vt_ref):
    x = x_ref[...]
    y = jnp.dot(x, w_ref[...], preferred_element_type=jnp.float32) + b_ref[...]
    q_ref[...] = y[:, :D]
    k_ref[...] = y[:, D:]
    ve = lax.dot_general(wve_ref[...], x, (((1,), (1,)), ((), ())),
                         preferred_element_type=jnp.float32) + bve_ref[...]
    vo = lax.dot_general(wvo_ref[...], x, (((1,), (1,)), ((), ())),
                         preferred_element_type=jnp.float32) + bvo_ref[...]
    lo = lax.shift_right_logical(_bf16_bits(ve), 16)
    hi = jnp.bitwise_and(_bf16_bits(vo), jnp.int32(-65536))
    vt_ref[...] = jnp.bitwise_or(lo, hi)


def _project(emb_pad, wqk, bqk, wve, wvo, bve, bvo):
    R = 512
    return pl.pallas_call(
        _proj_body,
        grid=(NP // R,),
        in_specs=[
            pl.BlockSpec((R, D), lambda i: (i, 0)),
            pl.BlockSpec((D, 2 * D), lambda i: (0, 0)),
            pl.BlockSpec((1, 2 * D), lambda i: (0, 0)),
            pl.BlockSpec((D // 2, D), lambda i: (0, 0)),
            pl.BlockSpec((D // 2, D), lambda i: (0, 0)),
            pl.BlockSpec((D // 2, 1), lambda i: (0, 0)),
            pl.BlockSpec((D // 2, 1), lambda i: (0, 0)),
        ],
        out_specs=[
            pl.BlockSpec((R, D), lambda i: (i, 0)),
            pl.BlockSpec((R, D), lambda i: (i, 0)),
            pl.BlockSpec((D // 2, R), lambda i: (0, i)),
        ],
        out_shape=[
            jax.ShapeDtypeStruct((NP, D), jnp.float32),
            jax.ShapeDtypeStruct((NP, D), jnp.float32),
            jax.ShapeDtypeStruct((D // 2, NP), jnp.int32),
        ],
    )(emb_pad, wqk, bqk, wve, wvo, bve, bvo)


EPW = NE_PAD // NW
C2 = 64
NCH2 = EPW // C2


def _scores_body(q_hbm, k_hbm, dst_hbm, src_hbm, s_hbm,
                 di_v, si_v, qrows, krows, sv, pacc, sem):
    cid = lax.axis_index("c")
    sid = lax.axis_index("s")
    base = (sid * NC + cid) * EPW

    def chunk(i, carry):
        off = base + i * C2
        pltpu.sync_copy(dst_hbm.at[pl.ds(off, C2)], di_v)
        pltpu.sync_copy(src_hbm.at[pl.ds(off, C2)], si_v)
        cp_q = pltpu.async_copy(q_hbm.at[di_v], qrows, sem)
        cp_k = pltpu.async_copy(k_hbm.at[si_v], krows, sem)
        cp_q.wait()
        cp_k.wait()
        lane = lax.iota(jnp.int32, L)

        def group(g, c):
            for e16 in range(L):
                e = g * L + e16
                acc = qrows[e, pl.ds(0, L)] * krows[e, pl.ds(0, L)]
                for j in range(1, D // L):
                    acc = acc + qrows[e, pl.ds(j * L, L)] * krows[e, pl.ds(j * L, L)]
                pacc[e16, :] = acc
            svec = plsc.load_gather(pacc, [lane, jnp.zeros((L,), jnp.int32)])
            for j in range(1, L):
                svec = svec + plsc.load_gather(pacc, [lane, jnp.full((L,), j, jnp.int32)])
            sv[pl.ds(g * L, L)] = svec
            return c

        lax.fori_loop(0, C2 // L, group, 0, unroll=False)
        pltpu.sync_copy(sv, s_hbm.at[pl.ds(off, C2)])
        return carry

    lax.fori_loop(0, NCH2, chunk, 0, unroll=False)


def _scores(qs, ks, dst, src):
    return pl.kernel(
        _scores_body,
        out_type=jax.ShapeDtypeStruct((NE_PAD,), jnp.float32),
        mesh=_sc_mesh(),
        compiler_params=_SC_PARAMS,
        scratch_types=[
            pltpu.VMEM((C2,), jnp.int32),
            pltpu.VMEM((C2,), jnp.int32),
            pltpu.VMEM((C2, D), jnp.float32),
            pltpu.VMEM((C2, D), jnp.float32),
            pltpu.VMEM((C2,), jnp.float32),
            pltpu.VMEM((L, L), jnp.float32),
            pltpu.SemaphoreType.DMA,
        ],
    )(qs, ks, dst, src)


SM_ROWS = NE_PAD // 128


def _softmax_body(s_ref, w_ref):
    s = s_ref[...]
    rows = lax.broadcasted_iota(jnp.int32, (SM_ROWS, 128), 0)
    cols = lax.broadcasted_iota(jnp.int32, (SM_ROWS, 128), 1)
    valid = rows * 128 + cols < N_EDGES
    s = jnp.where(valid, s, -jnp.inf)
    m = jnp.max(s)
    e = jnp.where(valid, jnp.exp(s - m), 0.0)
    w_ref[...] = e / jnp.sum(e)


def _softmax(scores):
    return pl.pallas_call(
        _softmax_body,
        out_shape=jax.ShapeDtypeStruct((SM_ROWS, 128), jnp.float32),
    )(scores.reshape(SM_ROWS, 128))


PAIRS = 4
CE = 800
NCH4 = N_EDGES // CE
GU = 10
M_HI = jnp.int32(-65536)


def _agg_body(vt_hbm, src_hbm, dst_hbm, w_hbm, agg_hbm,
              slab, acc, si0, di0, wv0, si1, di1, wv1, sem0, sem1, sems):
    cid = lax.axis_index("c")
    sid = lax.axis_index("s")
    wid = sid * NC + cid
    zero = jnp.zeros((L,), jnp.float32)
    bufs = ((si0, di0, wv0, sem0), (si1, di1, wv1, sem1))

    def issue(i, b):
        si, di, wv, sem = bufs[b]
        off = i * CE
        pltpu.async_copy(src_hbm.at[pl.ds(off, CE)], si, sem)
        pltpu.async_copy(dst_hbm.at[pl.ds(off, CE)], di, sem)
        pltpu.async_copy(w_hbm.at[pl.ds(off, CE)], wv, sem)

    def wait(b):
        si, di, wv, sem = bufs[b]
        pltpu.make_async_copy(src_hbm.at[pl.ds(0, CE)], si, sem).wait()
        pltpu.make_async_copy(dst_hbm.at[pl.ds(0, CE)], di, sem).wait()
        pltpu.make_async_copy(w_hbm.at[pl.ds(0, CE)], wv, sem).wait()

    def compute(b):
        si, di, wv, _ = bufs[b]

        def grp(i, c2):
            for u in range(GU):
                g = i * GU + u
                s16 = si[pl.ds(g * L, L)]
                d16 = di[pl.ds(g * L, L)]
                w16 = wv[pl.ds(g * L, L)]
                for j in range(PAIRS):
                    word = plsc.load_gather(slab, [s16 + (j * NP)])
                    fe = plsc.bitcast(lax.shift_left(word, 16), jnp.float32)
                    fo = plsc.bitcast(jnp.bitwise_and(word, M_HI), jnp.float32)
                    plsc.addupdate_scatter(acc, [d16 + (2 * j * NP)], fe * w16)
                    plsc.addupdate_scatter(acc, [d16 + ((2 * j + 1) * NP)], fo * w16)
            return c2

        lax.fori_loop(0, CE // (L * GU), grp, 0, unroll=False)

    cp_slab = pltpu.async_copy(vt_hbm.at[wid], slab, sems)
    issue(0, 0)

    def zinit(i, c):
        for j in range(2 * PAIRS):
            acc[pl.ds(j * NP + i * L, L)] = zero
        return c

    lax.fori_loop(0, NP // L, zinit, 0, unroll=False)
    cp_slab.wait()

    def outer(i2, c):
        ia = 2 * i2
        issue(ia + 1, 1)
        wait(0)
        compute(0)

        @pl.when(ia + 2 < NCH4)
        def _():
            issue(ia + 2, 0)

        wait(1)
        compute(1)
        return c

    lax.fori_loop(0, NCH4 // 2, outer, 0, unroll=False)
    pltpu.sync_copy(acc, agg_hbm.at[wid])


def _aggregate(vt_slabs, src, dst, w):
    return pl.kernel(
        _agg_body,
        out_type=jax.ShapeDtypeStruct((NW, 2 * PAIRS * NP), jnp.float32),
        mesh=_sc_mesh(),
        compiler_params=_SC_PARAMS,
        scratch_types=[
            pltpu.VMEM((PAIRS * NP,), jnp.int32),
            pltpu.VMEM((2 * PAIRS * NP,), jnp.float32),
            pltpu.VMEM((CE,), jnp.int32),
            pltpu.VMEM((CE,), jnp.int32),
            pltpu.VMEM((CE,), jnp.float32),
            pltpu.VMEM((CE,), jnp.int32),
            pltpu.VMEM((CE,), jnp.int32),
            pltpu.VMEM((CE,), jnp.float32),
            pltpu.SemaphoreType.DMA,
            pltpu.SemaphoreType.DMA,
            pltpu.SemaphoreType.DMA,
        ],
    )(vt_slabs, src, dst, w)


def _final_body(agg_ref, emb_ref, out_ref):
    out_ref[...] = emb_ref[...] + lax.transpose(agg_ref[...], (1, 0))


def _finalize(agg_t, emb_pad):
    R = 512
    return pl.pallas_call(
        _final_body,
        grid=(NP // R,),
        in_specs=[
            pl.BlockSpec((D, R), lambda i: (0, i)),
            pl.BlockSpec((R, D), lambda i: (i, 0)),
        ],
        out_specs=pl.BlockSpec((R, D), lambda i: (i, 0)),
        out_shape=jax.ShapeDtypeStruct((NP, D), jnp.float32),
    )(agg_t, emb_pad)


def kernel(embeddings, edge_index, Wq, bq, Wk, bk, Wv, bv):
    inv = 1.0 / (D ** 0.5)
    wqk = jnp.concatenate([Wq.T * inv, Wk.T], axis=1)
    bqk = jnp.concatenate([bq * inv, bk]).reshape(1, 2 * D)
    wve, wvo = Wv[0::2], Wv[1::2]
    bve, bvo = bv[0::2].reshape(D // 2, 1), bv[1::2].reshape(D // 2, 1)

    src = edge_index[0].astype(jnp.int32)
    dst = edge_index[1].astype(jnp.int32)
    src_p = jnp.pad(src, (0, NE_PAD - N_EDGES))
    dst_p = jnp.pad(dst, (0, NE_PAD - N_EDGES))
    emb_pad = jnp.pad(embeddings, ((0, NP - N_NODES), (0, 0)))

    qs, ks, vtp = _project(emb_pad, wqk, bqk, wve, wvo, bve, bvo)
    scores = _scores(qs, ks, dst_p, src_p)
    w = _softmax(scores).reshape(NE_PAD)

    vt_slabs = vtp.reshape(NW, PAIRS * NP)
    agg = _aggregate(vt_slabs, src, dst, w[:N_EDGES])
    out = _finalize(agg.reshape(D, NP), emb_pad)
    return out[:N_NODES]


# speedup vs baseline: 2.1218x; 1.3426x over previous
import jax
import jax.numpy as jnp
from jax import lax
from jax.experimental import pallas as pl
from jax.experimental.pallas import tpu as pltpu
from jax.experimental.pallas import tpu_sc as plsc

N_NODES = 10000
N_EDGES = 160000
D = 256
NC, NS, L = 2, 16, 16
NW = NC * NS
NP = 10240
NE_PAD = 163840

_SC_PARAMS = pltpu.CompilerParams(needs_layout_passes=False)


def _sc_mesh():
    return plsc.VectorSubcoreMesh(core_axis_name="c", subcore_axis_name="s",
                                  num_cores=NC, num_subcores=NS)


def _bf16_bits(x):
    return lax.bitcast_convert_type(x.astype(jnp.bfloat16).astype(jnp.float32),
                                    jnp.int32)


def _proj_body(x_ref, w_ref, b_ref, wve_ref, wvo_ref, bve_ref, bvo_ref,
               q_ref, k_ref, vt_ref):
    x = x_ref[...]
    y = jnp.dot(x, w_ref[...], preferred_element_type=jnp.float32) + b_ref[...]
    q_ref[...] = y[:, :D]
    k_ref[...] = y[:, D:]
    ve = lax.dot_general(wve_ref[...], x, (((1,), (1,)), ((), ())),
                         preferred_element_type=jnp.float32) + bve_ref[...]
    vo = lax.dot_general(wvo_ref[...], x, (((1,), (1,)), ((), ())),
                         preferred_element_type=jnp.float32) + bvo_ref[...]
    lo = lax.shift_right_logical(_bf16_bits(ve), 16)
    hi = jnp.bitwise_and(_bf16_bits(vo), jnp.int32(-65536))
    vt_ref[...] = jnp.bitwise_or(lo, hi)


def _project(emb_pad, wqk, bqk, wve, wvo, bve, bvo):
    R = 512
    return pl.pallas_call(
        _proj_body,
        grid=(NP // R,),
        in_specs=[
            pl.BlockSpec((R, D), lambda i: (i, 0)),
            pl.BlockSpec((D, 2 * D), lambda i: (0, 0)),
            pl.BlockSpec((1, 2 * D), lambda i: (0, 0)),
            pl.BlockSpec((D // 2, D), lambda i: (0, 0)),
            pl.BlockSpec((D // 2, D), lambda i: (0, 0)),
            pl.BlockSpec((D // 2, 1), lambda i: (0, 0)),
            pl.BlockSpec((D // 2, 1), lambda i: (0, 0)),
        ],
        out_specs=[
            pl.BlockSpec((R, D), lambda i: (i, 0)),
            pl.BlockSpec((R, D), lambda i: (i, 0)),
            pl.BlockSpec((D // 2, R), lambda i: (0, i)),
        ],
        out_shape=[
            jax.ShapeDtypeStruct((NP, D), jnp.float32),
            jax.ShapeDtypeStruct((NP, D), jnp.float32),
            jax.ShapeDtypeStruct((D // 2, NP), jnp.int32),
        ],
    )(emb_pad, wqk, bqk, wve, wvo, bve, bvo)


EPW = NE_PAD // NW
C2 = 64
NCH2 = EPW // C2


def _scores_body(q_hbm, k_hbm, dst_hbm, src_hbm, s_hbm,
                 di_v, si_v, qr0, kr0, qr1, kr1, sv, pacc, semi, sem0, sem1):
    cid = lax.axis_index("c")
    sid = lax.axis_index("s")
    base = (sid * NC + cid) * EPW
    rbufs = ((qr0, kr0, sem0), (qr1, kr1, sem1))

    cpd = pltpu.async_copy(dst_hbm.at[pl.ds(base, EPW)], di_v, semi)
    cps = pltpu.async_copy(src_hbm.at[pl.ds(base, EPW)], si_v, semi)
    cpd.wait()
    cps.wait()

    def issue_rows(i, b):
        qr, kr, sem = rbufs[b]
        pltpu.async_copy(q_hbm.at[di_v.at[pl.ds(i * C2, C2)]], qr, sem)
        pltpu.async_copy(k_hbm.at[si_v.at[pl.ds(i * C2, C2)]], kr, sem)

    def wait_rows(b):
        qr, kr, sem = rbufs[b]
        pltpu.make_async_copy(q_hbm.at[pl.ds(0, C2)], qr, sem).wait()
        pltpu.make_async_copy(k_hbm.at[pl.ds(0, C2)], kr, sem).wait()

    lane = lax.iota(jnp.int32, L)

    def compute(i, b):
        qr, kr, _ = rbufs[b]

        def group(g, c):
            for e16 in range(L):
                e = g * L + e16
                acc = qr[e, pl.ds(0, L)] * kr[e, pl.ds(0, L)]
                for j in range(1, D // L):
                    acc = acc + qr[e, pl.ds(j * L, L)] * kr[e, pl.ds(j * L, L)]
                pacc[e16, :] = acc
            svec = plsc.load_gather(pacc, [lane, jnp.zeros((L,), jnp.int32)])
            for j in range(1, L):
                svec = svec + plsc.load_gather(pacc, [lane, jnp.full((L,), j, jnp.int32)])
            sv[pl.ds(i * C2 + g * L, L)] = svec
            return c

        lax.fori_loop(0, C2 // L, group, 0, unroll=False)

    issue_rows(0, 0)

    def outer(i2, c):
        a = 2 * i2
        issue_rows(a + 1, 1)
        wait_rows(0)
        compute(a, 0)

        @pl.when(a + 2 < NCH2)
        def _():
            issue_rows(a + 2, 0)

        wait_rows(1)
        compute(a + 1, 1)
        return c

    lax.fori_loop(0, NCH2 // 2, outer, 0, unroll=False)
    pltpu.sync_copy(sv, s_hbm.at[pl.ds(base, EPW)])


def _scores(qs, ks, dst, src):
    return pl.kernel(
        _scores_body,
        out_type=jax.ShapeDtypeStruct((NE_PAD,), jnp.float32),
        mesh=_sc_mesh(),
        compiler_params=_SC_PARAMS,
        scratch_types=[
            pltpu.VMEM((EPW,), jnp.int32),
            pltpu.VMEM((EPW,), jnp.int32),
            pltpu.VMEM((C2, D), jnp.float32),
            pltpu.VMEM((C2, D), jnp.float32),
            pltpu.VMEM((C2, D), jnp.float32),
            pltpu.VMEM((C2, D), jnp.float32),
            pltpu.VMEM((EPW,), jnp.float32),
            pltpu.VMEM((L, L), jnp.float32),
            pltpu.SemaphoreType.DMA,
            pltpu.SemaphoreType.DMA,
            pltpu.SemaphoreType.DMA,
        ],
    )(qs, ks, dst, src)


SM_ROWS = NE_PAD // 128


def _softmax_body(s_ref, w_ref):
    s = s_ref[...]
    rows = lax.broadcasted_iota(jnp.int32, (SM_ROWS, 128), 0)
    cols = lax.broadcasted_iota(jnp.int32, (SM_ROWS, 128), 1)
    valid = rows * 128 + cols < N_EDGES
    s = jnp.where(valid, s, -jnp.inf)
    m = jnp.max(s)
    e = jnp.where(valid, jnp.exp(s - m), 0.0)
    w_ref[...] = e / jnp.sum(e)


def _softmax(scores):
    return pl.pallas_call(
        _softmax_body,
        out_shape=jax.ShapeDtypeStruct((SM_ROWS, 128), jnp.float32),
    )(scores.reshape(SM_ROWS, 128))


PAIRS = 4
CE = 800
NCH4 = N_EDGES // CE
GU = 10
M_HI = -65536


def _agg_body(vt_hbm, src_hbm, dst_hbm, w_hbm, agg_hbm,
              slab, acc, si0, di0, wv0, si1, di1, wv1, sem0, sem1, sems):
    cid = lax.axis_index("c")
    sid = lax.axis_index("s")
    wid = sid * NC + cid
    zero = jnp.zeros((L,), jnp.float32)
    bufs = ((si0, di0, wv0, sem0), (si1, di1, wv1, sem1))

    def issue(i, b):
        si, di, wv, sem = bufs[b]
        off = i * CE
        pltpu.async_copy(src_hbm.at[pl.ds(off, CE)], si, sem)
        pltpu.async_copy(dst_hbm.at[pl.ds(off, CE)], di, sem)
        pltpu.async_copy(w_hbm.at[pl.ds(off, CE)], wv, sem)

    def wait(b):
        si, di, wv, sem = bufs[b]
        pltpu.make_async_copy(src_hbm.at[pl.ds(0, CE)], si, sem).wait()
        pltpu.make_async_copy(dst_hbm.at[pl.ds(0, CE)], di, sem).wait()
        pltpu.make_async_copy(w_hbm.at[pl.ds(0, CE)], wv, sem).wait()

    def compute(b):
        si, di, wv, _ = bufs[b]

        def grp(i, c2):
            for u in range(GU):
                g = i * GU + u
                s16 = si[pl.ds(g * L, L)]
                d16 = di[pl.ds(g * L, L)]
                w16 = wv[pl.ds(g * L, L)]
                for j in range(PAIRS):
                    word = plsc.load_gather(slab, [s16 + (j * NP)])
                    fe = plsc.bitcast(lax.shift_left(word, 16), jnp.float32)
                    fo = plsc.bitcast(jnp.bitwise_and(word, jnp.int32(M_HI)), jnp.float32)
                    plsc.addupdate_scatter(acc, [d16 + (2 * j * NP)], fe * w16)
                    plsc.addupdate_scatter(acc, [d16 + ((2 * j + 1) * NP)], fo * w16)
            return c2

        lax.fori_loop(0, CE // (L * GU), grp, 0, unroll=False)

    cp_slab = pltpu.async_copy(vt_hbm.at[wid], slab, sems)
    issue(0, 0)

    def zinit(i, c):
        for j in range(2 * PAIRS):
            acc[pl.ds(j * NP + i * L, L)] = zero
        return c

    lax.fori_loop(0, NP // L, zinit, 0, unroll=False)
    cp_slab.wait()

    def outer(i2, c):
        ia = 2 * i2
        issue(ia + 1, 1)
        wait(0)
        compute(0)

        @pl.when(ia + 2 < NCH4)
        def _():
            issue(ia + 2, 0)

        wait(1)
        compute(1)
        return c

    lax.fori_loop(0, NCH4 // 2, outer, 0, unroll=False)
    pltpu.sync_copy(acc, agg_hbm.at[wid])


def _aggregate(vt_slabs, src, dst, w):
    return pl.kernel(
        _agg_body,
        out_type=jax.ShapeDtypeStruct((NW, 2 * PAIRS * NP), jnp.float32),
        mesh=_sc_mesh(),
        compiler_params=_SC_PARAMS,
        scratch_types=[
            pltpu.VMEM((PAIRS * NP,), jnp.int32),
            pltpu.VMEM((2 * PAIRS * NP,), jnp.float32),
            pltpu.VMEM((CE,), jnp.int32),
            pltpu.VMEM((CE,), jnp.int32),
            pltpu.VMEM((CE,), jnp.float32),
            pltpu.VMEM((CE,), jnp.int32),
            pltpu.VMEM((CE,), jnp.int32),
            pltpu.VMEM((CE,), jnp.float32),
            pltpu.SemaphoreType.DMA,
            pltpu.SemaphoreType.DMA,
            pltpu.SemaphoreType.DMA,
        ],
    )(vt_slabs, src, dst, w)


def _final_body(agg_ref, emb_ref, out_ref):
    out_ref[...] = emb_ref[...] + lax.transpose(agg_ref[...], (1, 0))


def _finalize(agg_t, emb_pad):
    R = 512
    return pl.pallas_call(
        _final_body,
        grid=(NP // R,),
        in_specs=[
            pl.BlockSpec((D, R), lambda i: (0, i)),
            pl.BlockSpec((R, D), lambda i: (i, 0)),
        ],
        out_specs=pl.BlockSpec((R, D), lambda i: (i, 0)),
        out_shape=jax.ShapeDtypeStruct((NP, D), jnp.float32),
    )(agg_t, emb_pad)


def kernel(embeddings, edge_index, Wq, bq, Wk, bk, Wv, bv):
    inv = 1.0 / (D ** 0.5)
    wqk = jnp.concatenate([Wq.T * inv, Wk.T], axis=1)
    bqk = jnp.concatenate([bq * inv, bk]).reshape(1, 2 * D)
    wve, wvo = Wv[0::2], Wv[1::2]
    bve, bvo = bv[0::2].reshape(D // 2, 1), bv[1::2].reshape(D // 2, 1)

    src = edge_index[0].astype(jnp.int32)
    dst = edge_index[1].astype(jnp.int32)
    src_p = jnp.pad(src, (0, NE_PAD - N_EDGES))
    dst_p = jnp.pad(dst, (0, NE_PAD - N_EDGES))
    emb_pad = jnp.pad(embeddings, ((0, NP - N_NODES), (0, 0)))

    qs, ks, vtp = _project(emb_pad, wqk, bqk, wve, wvo, bve, bvo)
    scores = _scores(qs, ks, dst_p, src_p)
    w = _softmax(scores).reshape(NE_PAD)

    vt_slabs = vtp.reshape(NW, PAIRS * NP)
    agg = _aggregate(vt_slabs, src, dst, w[:N_EDGES])
    out = _finalize(agg.reshape(D, NP), emb_pad)
    return out[:N_NODES]


# speedup vs baseline: 2.2624x; 1.0662x over previous
import jax
import jax.numpy as jnp
from jax import lax
from jax.experimental import pallas as pl
from jax.experimental.pallas import tpu as pltpu
from jax.experimental.pallas import tpu_sc as plsc

N_NODES = 10000
N_EDGES = 160000
D = 256
NC, NS, L = 2, 16, 16
NW = NC * NS
NP = 10240
NE_PAD = 163840

_SC_PARAMS = pltpu.CompilerParams(needs_layout_passes=False)


def _sc_mesh():
    return plsc.VectorSubcoreMesh(core_axis_name="c", subcore_axis_name="s",
                                  num_cores=NC, num_subcores=NS)


def _bf16_bits(x):
    return lax.bitcast_convert_type(x.astype(jnp.bfloat16).astype(jnp.float32),
                                    jnp.int32)


def _proj_body(x_ref, wq_ref, wk_ref, wve_ref, wvo_ref,
               bq_ref, bk_ref, bve_ref, bvo_ref,
               q_ref, k_ref, vt_ref):
    x = x_ref[...]
    inv = 1.0 / (D ** 0.5)
    q = lax.dot_general(x, wq_ref[...], (((1,), (1,)), ((), ())),
                        preferred_element_type=jnp.float32)
    q_ref[...] = (q + bq_ref[...]) * inv
    k = lax.dot_general(x, wk_ref[...], (((1,), (1,)), ((), ())),
                        preferred_element_type=jnp.float32)
    k_ref[...] = k + bk_ref[...]
    ve = lax.dot_general(wve_ref[...], x, (((1,), (1,)), ((), ())),
                         preferred_element_type=jnp.float32) + bve_ref[...]
    vo = lax.dot_general(wvo_ref[...], x, (((1,), (1,)), ((), ())),
                         preferred_element_type=jnp.float32) + bvo_ref[...]
    lo = lax.shift_right_logical(_bf16_bits(ve), 16)
    hi = jnp.bitwise_and(_bf16_bits(vo), jnp.int32(-65536))
    vt_ref[...] = jnp.bitwise_or(lo, hi)


def _project(emb, wq, wk, wve, wvo, bqr, bkr, bve, bvo):
    R = 512
    return pl.pallas_call(
        _proj_body,
        grid=(NP // R,),
        in_specs=[
            pl.BlockSpec((R, D), lambda i: (i, 0)),
            pl.BlockSpec((D, D), lambda i: (0, 0)),
            pl.BlockSpec((D, D), lambda i: (0, 0)),
            pl.BlockSpec((D // 2, D), lambda i: (0, 0)),
            pl.BlockSpec((D // 2, D), lambda i: (0, 0)),
            pl.BlockSpec((1, D), lambda i: (0, 0)),
            pl.BlockSpec((1, D), lambda i: (0, 0)),
            pl.BlockSpec((D // 2, 1), lambda i: (0, 0)),
            pl.BlockSpec((D // 2, 1), lambda i: (0, 0)),
        ],
        out_specs=[
            pl.BlockSpec((R, D), lambda i: (i, 0)),
            pl.BlockSpec((R, D), lambda i: (i, 0)),
            pl.BlockSpec((D // 2, R), lambda i: (0, i)),
        ],
        out_shape=[
            jax.ShapeDtypeStruct((NP, D), jnp.float32),
            jax.ShapeDtypeStruct((NP, D), jnp.float32),
            jax.ShapeDtypeStruct((D // 2, NP), jnp.int32),
        ],
    )(emb, wq, wk, wve, wvo, bqr, bkr, bve, bvo)


EROWS = N_EDGES // 128
EROWS_P = NE_PAD // 128


def _pad_body(e_ref, s_ref, d_ref):
    z = jnp.zeros((EROWS_P - EROWS, 128), jnp.int32)
    s_ref[...] = jnp.concatenate([e_ref[0], z], axis=0)
    d_ref[...] = jnp.concatenate([e_ref[1], z], axis=0)


def _pad_edges(edge_index):
    return pl.pallas_call(
        _pad_body,
        out_shape=[
            jax.ShapeDtypeStruct((EROWS_P, 128), jnp.int32),
            jax.ShapeDtypeStruct((EROWS_P, 128), jnp.int32),
        ],
    )(edge_index.reshape(2, EROWS, 128))


EPW = NE_PAD // NW
C2 = 64
NCH2 = EPW // C2


def _scores_body(q_hbm, k_hbm, dst_hbm, src_hbm, s_hbm,
                 di_v, si_v, qr0, kr0, qr1, kr1, sv, pacc, semi, sem0, sem1):
    cid = lax.axis_index("c")
    sid = lax.axis_index("s")
    base = (sid * NC + cid) * EPW
    rbufs = ((qr0, kr0, sem0), (qr1, kr1, sem1))

    cpd = pltpu.async_copy(dst_hbm.at[pl.ds(base, EPW)], di_v, semi)
    cps = pltpu.async_copy(src_hbm.at[pl.ds(base, EPW)], si_v, semi)
    cpd.wait()
    cps.wait()

    def issue_rows(i, b):
        qr, kr, sem = rbufs[b]
        pltpu.async_copy(q_hbm.at[di_v.at[pl.ds(i * C2, C2)]], qr, sem)
        pltpu.async_copy(k_hbm.at[si_v.at[pl.ds(i * C2, C2)]], kr, sem)

    def wait_rows(b):
        qr, kr, sem = rbufs[b]
        pltpu.make_async_copy(q_hbm.at[pl.ds(0, C2)], qr, sem).wait()
        pltpu.make_async_copy(k_hbm.at[pl.ds(0, C2)], kr, sem).wait()

    lane = lax.iota(jnp.int32, L)

    def compute(i, b):
        qr, kr, _ = rbufs[b]

        def group(g, c):
            for e16 in range(L):
                e = g * L + e16
                acc = qr[e, pl.ds(0, L)] * kr[e, pl.ds(0, L)]
                for j in range(1, D // L):
                    acc = acc + qr[e, pl.ds(j * L, L)] * kr[e, pl.ds(j * L, L)]
                pacc[e16, :] = acc
            svec = plsc.load_gather(pacc, [lane, jnp.zeros((L,), jnp.int32)])
            for j in range(1, L):
                svec = svec + plsc.load_gather(pacc, [lane, jnp.full((L,), j, jnp.int32)])
            sv[pl.ds(i * C2 + g * L, L)] = svec
            return c

        lax.fori_loop(0, C2 // L, group, 0, unroll=False)

    issue_rows(0, 0)

    def outer(i2, c):
        a = 2 * i2
        issue_rows(a + 1, 1)
        wait_rows(0)
        compute(a, 0)

        @pl.when(a + 2 < NCH2)
        def _():
            issue_rows(a + 2, 0)

        wait_rows(1)
        compute(a + 1, 1)
        return c

    lax.fori_loop(0, NCH2 // 2, outer, 0, unroll=False)
    pltpu.sync_copy(sv, s_hbm.at[pl.ds(base, EPW)])


def _scores(qs, ks, dst, src):
    return pl.kernel(
        _scores_body,
        out_type=jax.ShapeDtypeStruct((NE_PAD,), jnp.float32),
        mesh=_sc_mesh(),
        compiler_params=_SC_PARAMS,
        scratch_types=[
            pltpu.VMEM((EPW,), jnp.int32),
            pltpu.VMEM((EPW,), jnp.int32),
            pltpu.VMEM((C2, D), jnp.float32),
            pltpu.VMEM((C2, D), jnp.float32),
            pltpu.VMEM((C2, D), jnp.float32),
            pltpu.VMEM((C2, D), jnp.float32),
            pltpu.VMEM((EPW,), jnp.float32),
            pltpu.VMEM((L, L), jnp.float32),
            pltpu.SemaphoreType.DMA,
            pltpu.SemaphoreType.DMA,
            pltpu.SemaphoreType.DMA,
        ],
    )(qs, ks, dst, src)


SM_ROWS = NE_PAD // 128


def _softmax_body(s_ref, w_ref):
    s = s_ref[...]
    rows = lax.broadcasted_iota(jnp.int32, (SM_ROWS, 128), 0)
    cols = lax.broadcasted_iota(jnp.int32, (SM_ROWS, 128), 1)
    valid = rows * 128 + cols < N_EDGES
    s = jnp.where(valid, s, -jnp.inf)
    m = jnp.max(s)
    e = jnp.where(valid, jnp.exp(s - m), 0.0)
    w_ref[...] = e / jnp.sum(e)


def _softmax(scores):
    return pl.pallas_call(
        _softmax_body,
        out_shape=jax.ShapeDtypeStruct((SM_ROWS, 128), jnp.float32),
    )(scores.reshape(SM_ROWS, 128))


PAIRS = 4
CE = 800
NCH4 = N_EDGES // CE
GU = 10
M_HI = -65536


def _agg_body(vt_hbm, src_hbm, dst_hbm, w_hbm, agg_hbm,
              slab, acc, si0, di0, wv0, si1, di1, wv1, sem0, sem1, sems):
    cid = lax.axis_index("c")
    sid = lax.axis_index("s")
    wid = sid * NC + cid
    zero = jnp.zeros((L,), jnp.float32)
    bufs = ((si0, di0, wv0, sem0), (si1, di1, wv1, sem1))

    def issue(i, b):
        si, di, wv, sem = bufs[b]
        off = i * CE
        pltpu.async_copy(src_hbm.at[pl.ds(off, CE)], si, sem)
        pltpu.async_copy(dst_hbm.at[pl.ds(off, CE)], di, sem)
        pltpu.async_copy(w_hbm.at[pl.ds(off, CE)], wv, sem)

    def wait(b):
        si, di, wv, sem = bufs[b]
        pltpu.make_async_copy(src_hbm.at[pl.ds(0, CE)], si, sem).wait()
        pltpu.make_async_copy(dst_hbm.at[pl.ds(0, CE)], di, sem).wait()
        pltpu.make_async_copy(w_hbm.at[pl.ds(0, CE)], wv, sem).wait()

    def compute(b):
        si, di, wv, _ = bufs[b]

        def grp(i, c2):
            for u in range(GU):
                g = i * GU + u
                s16 = si[pl.ds(g * L, L)]
                d16 = di[pl.ds(g * L, L)]
                w16 = wv[pl.ds(g * L, L)]
                for j in range(PAIRS):
                    word = plsc.load_gather(slab, [s16 + (j * NP)])
                    fe = plsc.bitcast(lax.shift_left(word, 16), jnp.float32)
                    fo = plsc.bitcast(jnp.bitwise_and(word, jnp.int32(M_HI)), jnp.float32)
                    plsc.addupdate_scatter(acc, [d16 + (2 * j * NP)], fe * w16)
                    plsc.addupdate_scatter(acc, [d16 + ((2 * j + 1) * NP)], fo * w16)
            return c2

        lax.fori_loop(0, CE // (L * GU), grp, 0, unroll=False)

    cp_slab = pltpu.async_copy(vt_hbm.at[wid], slab, sems)
    issue(0, 0)

    def zinit(i, c):
        for j in range(2 * PAIRS):
            acc[pl.ds(j * NP + i * L, L)] = zero
        return c

    lax.fori_loop(0, NP // L, zinit, 0, unroll=False)
    cp_slab.wait()

    def outer(i2, c):
        ia = 2 * i2
        issue(ia + 1, 1)
        wait(0)
        compute(0)

        @pl.when(ia + 2 < NCH4)
        def _():
            issue(ia + 2, 0)

        wait(1)
        compute(1)
        return c

    lax.fori_loop(0, NCH4 // 2, outer, 0, unroll=False)
    pltpu.sync_copy(acc, agg_hbm.at[wid])


def _aggregate(vt_slabs, src, dst, w):
    return pl.kernel(
        _agg_body,
        out_type=jax.ShapeDtypeStruct((NW, 2 * PAIRS * NP), jnp.float32),
        mesh=_sc_mesh(),
        compiler_params=_SC_PARAMS,
        scratch_types=[
            pltpu.VMEM((PAIRS * NP,), jnp.int32),
            pltpu.VMEM((2 * PAIRS * NP,), jnp.float32),
            pltpu.VMEM((CE,), jnp.int32),
            pltpu.VMEM((CE,), jnp.int32),
            pltpu.VMEM((CE,), jnp.float32),
            pltpu.VMEM((CE,), jnp.int32),
            pltpu.VMEM((CE,), jnp.int32),
            pltpu.VMEM((CE,), jnp.float32),
            pltpu.SemaphoreType.DMA,
            pltpu.SemaphoreType.DMA,
            pltpu.SemaphoreType.DMA,
        ],
    )(vt_slabs, src, dst, w)


def _final_body(agg_ref, emb_ref, out_ref):
    out_ref[...] = emb_ref[...] + lax.transpose(agg_ref[...], (1, 0))


def _finalize(agg_t, emb):
    R = 512
    return pl.pallas_call(
        _final_body,
        grid=(NP // R,),
        in_specs=[
            pl.BlockSpec((D, R), lambda i: (0, i)),
            pl.BlockSpec((R, D), lambda i: (i, 0)),
        ],
        out_specs=pl.BlockSpec((R, D), lambda i: (i, 0)),
        out_shape=jax.ShapeDtypeStruct((N_NODES, D), jnp.float32),
    )(agg_t, emb)


def kernel(embeddings, edge_index, Wq, bq, Wk, bk, Wv, bv):
    wve, wvo = Wv[0::2], Wv[1::2]
    bve, bvo = bv[0::2].reshape(D // 2, 1), bv[1::2].reshape(D // 2, 1)
    bqr, bkr = bq.reshape(1, D), bk.reshape(1, D)

    src_p, dst_p = _pad_edges(edge_index.astype(jnp.int32))
    src_p = src_p.reshape(NE_PAD)
    dst_p = dst_p.reshape(NE_PAD)

    qs, ks, vtp = _project(embeddings, Wq, Wk, wve, wvo, bqr, bkr, bve, bvo)
    scores = _scores(qs, ks, dst_p, src_p)
    w = _softmax(scores).reshape(NE_PAD)

    vt_slabs = vtp.reshape(NW, PAIRS * NP)
    agg = _aggregate(vt_slabs, src_p, dst_p, w)
    return _finalize(agg.reshape(D, NP), embeddings)


# speedup vs baseline: 2.5468x; 1.1257x over previous
import jax
import jax.numpy as jnp
from jax import lax
from jax.experimental import pallas as pl
from jax.experimental.pallas import tpu as pltpu
from jax.experimental.pallas import tpu_sc as plsc

N_NODES = 10000
N_EDGES = 160000
D = 256
NC, NS, L = 2, 16, 16
NW = NC * NS
NP = 10240
NE_PAD = 163840

_SC_PARAMS = pltpu.CompilerParams(needs_layout_passes=False)


def _sc_mesh():
    return plsc.VectorSubcoreMesh(core_axis_name="c", subcore_axis_name="s",
                                  num_cores=NC, num_subcores=NS)


def _bf16_bits(x):
    return lax.bitcast_convert_type(x.astype(jnp.bfloat16).astype(jnp.float32),
                                    jnp.int32)


def _pack_pair(even, odd):
    lo = lax.shift_right_logical(_bf16_bits(even), 16)
    hi = jnp.bitwise_and(_bf16_bits(odd), jnp.int32(-65536))
    return jnp.bitwise_or(lo, hi)


def _proj_body(x_ref, wqe_ref, wqo_ref, wke_ref, wko_ref, wve_ref, wvo_ref,
               bqe_ref, bqo_ref, bke_ref, bko_ref, bve_ref, bvo_ref,
               qt_ref, kt_ref, vt_ref):
    x = x_ref[...]
    inv = 1.0 / (D ** 0.5)
    dn = (((1,), (1,)), ((), ()))

    def mm(w_ref, b_ref):
        return lax.dot_general(w_ref[...], x, dn,
                               preferred_element_type=jnp.float32) + b_ref[...]

    qt_ref[...] = _pack_pair(mm(wqe_ref, bqe_ref) * inv, mm(wqo_ref, bqo_ref) * inv)
    kt_ref[...] = _pack_pair(mm(wke_ref, bke_ref), mm(wko_ref, bko_ref))
    vt_ref[...] = _pack_pair(mm(wve_ref, bve_ref), mm(wvo_ref, bvo_ref))


def _project(emb, ws, bs):
    R = 512
    half = pl.BlockSpec((D // 2, D), lambda i: (0, 0))
    bcol = pl.BlockSpec((D // 2, 1), lambda i: (0, 0))
    outs = pl.BlockSpec((D // 2, R), lambda i: (0, i))
    return pl.pallas_call(
        _proj_body,
        grid=(NP // R,),
        in_specs=[pl.BlockSpec((R, D), lambda i: (i, 0))] + [half] * 6 + [bcol] * 6,
        out_specs=[outs] * 3,
        out_shape=[jax.ShapeDtypeStruct((D // 2, NP), jnp.int32)] * 3,
    )(emb, *ws, *bs)


EROWS = N_EDGES // 128
EROWS_P = NE_PAD // 128


def _pad_body(e_ref, s_ref, d_ref):
    z = jnp.zeros((EROWS_P - EROWS, 128), jnp.int32)
    s_ref[...] = jnp.concatenate([e_ref[0], z], axis=0)
    d_ref[...] = jnp.concatenate([e_ref[1], z], axis=0)


def _pad_edges(edge_index):
    return pl.pallas_call(
        _pad_body,
        out_shape=[
            jax.ShapeDtypeStruct((EROWS_P, 128), jnp.int32),
            jax.ShapeDtypeStruct((EROWS_P, 128), jnp.int32),
        ],
    )(edge_index.reshape(2, EROWS, 128))


C2 = 2048
NCH2 = NE_PAD // C2
GU2 = 8


def _scores_body(qt_hbm, kt_hbm, dst_hbm, src_hbm, sp_hbm,
                 qslab, kslab, si0, di0, si1, di1, pb0, pb1,
                 semq, sem0, sem1, semp):
    cid = lax.axis_index("c")
    sid = lax.axis_index("s")
    wid = sid * NC + cid
    ibufs = ((si0, di0, sem0), (si1, di1, sem1))
    pbufs = (pb0, pb1)

    def issue(i, b):
        si, di, sem = ibufs[b]
        off = i * C2
        pltpu.async_copy(src_hbm.at[pl.ds(off, C2)], si, sem)
        pltpu.async_copy(dst_hbm.at[pl.ds(off, C2)], di, sem)

    def wait(b):
        si, di, sem = ibufs[b]
        pltpu.make_async_copy(src_hbm.at[pl.ds(0, C2)], si, sem).wait()
        pltpu.make_async_copy(dst_hbm.at[pl.ds(0, C2)], di, sem).wait()

    def compute(i, b):
        si, di, _ = ibufs[b]
        pbuf = pbufs[b]

        def grp(t, c2):
            for u in range(GU2):
                g = t * GU2 + u
                s16 = si[pl.ds(g * L, L)]
                d16 = di[pl.ds(g * L, L)]
                acc = jnp.zeros((L,), jnp.float32)
                for j in range(PAIRS):
                    qw = plsc.load_gather(qslab, [d16 + (j * NP)])
                    kw = plsc.load_gather(kslab, [s16 + (j * NP)])
                    qe = plsc.bitcast(lax.shift_left(qw, 16), jnp.float32)
                    ke = plsc.bitcast(lax.shift_left(kw, 16), jnp.float32)
                    qo = plsc.bitcast(jnp.bitwise_and(qw, jnp.int32(M_HI)), jnp.float32)
                    ko = plsc.bitcast(jnp.bitwise_and(kw, jnp.int32(M_HI)), jnp.float32)
                    acc = acc + qe * ke + qo * ko
                pbuf[pl.ds(g * L, L)] = acc
            return c2

        lax.fori_loop(0, C2 // (L * GU2), grp, 0, unroll=False)
        pltpu.async_copy(pbuf, sp_hbm.at[wid, pl.ds(i * C2, C2)], semp)

    def wait_pbuf(b):
        pltpu.make_async_copy(pbufs[b], sp_hbm.at[wid, pl.ds(0, C2)], semp).wait()

    cpq = pltpu.async_copy(qt_hbm.at[wid], qslab, semq)
    cpk = pltpu.async_copy(kt_hbm.at[wid], kslab, semq)
    issue(0, 0)
    cpq.wait()
    cpk.wait()

    def outer(i2, c):
        a = 2 * i2
        issue(a + 1, 1)
        wait(0)

        @pl.when(i2 > 0)
        def _():
            wait_pbuf(0)

        compute(a, 0)

        @pl.when(a + 2 < NCH2)
        def _():
            issue(a + 2, 0)

        wait(1)

        @pl.when(i2 > 0)
        def _():
            wait_pbuf(1)

        compute(a + 1, 1)
        return c

    lax.fori_loop(0, NCH2 // 2, outer, 0, unroll=False)
    wait_pbuf(0)
    wait_pbuf(1)


def _scores(qt_slabs, kt_slabs, dst, src):
    return pl.kernel(
        _scores_body,
        out_type=jax.ShapeDtypeStruct((NW, NE_PAD), jnp.float32),
        mesh=_sc_mesh(),
        compiler_params=_SC_PARAMS,
        scratch_types=[
            pltpu.VMEM((PAIRS * NP,), jnp.int32),
            pltpu.VMEM((PAIRS * NP,), jnp.int32),
            pltpu.VMEM((C2,), jnp.int32),
            pltpu.VMEM((C2,), jnp.int32),
            pltpu.VMEM((C2,), jnp.int32),
            pltpu.VMEM((C2,), jnp.int32),
            pltpu.VMEM((C2,), jnp.float32),
            pltpu.VMEM((C2,), jnp.float32),
            pltpu.SemaphoreType.DMA,
            pltpu.SemaphoreType.DMA,
            pltpu.SemaphoreType.DMA,
            pltpu.SemaphoreType.DMA,
        ],
    )(qt_slabs, kt_slabs, dst, src)


SM_ROWS = NE_PAD // 128


def _softmax_body(sp_ref, w_ref):
    s = jnp.sum(sp_ref[...], axis=0)
    rows = lax.broadcasted_iota(jnp.int32, (SM_ROWS, 128), 0)
    cols = lax.broadcasted_iota(jnp.int32, (SM_ROWS, 128), 1)
    valid = rows * 128 + cols < N_EDGES
    s = jnp.where(valid, s, -jnp.inf)
    m = jnp.max(s)
    e = jnp.where(valid, jnp.exp(s - m), 0.0)
    w_ref[...] = e / jnp.sum(e)


def _softmax(sparts):
    return pl.pallas_call(
        _softmax_body,
        out_shape=jax.ShapeDtypeStruct((SM_ROWS, 128), jnp.float32),
    )(sparts.reshape(NW, SM_ROWS, 128))


PAIRS = 4
CE = 800
NCH4 = N_EDGES // CE
GU = 10
M_HI = -65536


def _agg_body(vt_hbm, src_hbm, dst_hbm, w_hbm, agg_hbm,
              slab, acc, si0, di0, wv0, si1, di1, wv1, sem0, sem1, sems):
    cid = lax.axis_index("c")
    sid = lax.axis_index("s")
    wid = sid * NC + cid
    zero = jnp.zeros((L,), jnp.float32)
    bufs = ((si0, di0, wv0, sem0), (si1, di1, wv1, sem1))

    def issue(i, b):
        si, di, wv, sem = bufs[b]
        off = i * CE
        pltpu.async_copy(src_hbm.at[pl.ds(off, CE)], si, sem)
        pltpu.async_copy(dst_hbm.at[pl.ds(off, CE)], di, sem)
        pltpu.async_copy(w_hbm.at[pl.ds(off, CE)], wv, sem)

    def wait(b):
        si, di, wv, sem = bufs[b]
        pltpu.make_async_copy(src_hbm.at[pl.ds(0, CE)], si, sem).wait()
        pltpu.make_async_copy(dst_hbm.at[pl.ds(0, CE)], di, sem).wait()
        pltpu.make_async_copy(w_hbm.at[pl.ds(0, CE)], wv, sem).wait()

    def compute(b):
        si, di, wv, _ = bufs[b]

        def grp(i, c2):
            for u in range(GU):
                g = i * GU + u
                s16 = si[pl.ds(g * L, L)]
                d16 = di[pl.ds(g * L, L)]
                w16 = wv[pl.ds(g * L, L)]
                for j in range(PAIRS):
                    word = plsc.load_gather(slab, [s16 + (j * NP)])
                    fe = plsc.bitcast(lax.shift_left(word, 16), jnp.float32)
                    fo = plsc.bitcast(jnp.bitwise_and(word, jnp.int32(M_HI)), jnp.float32)
                    plsc.addupdate_scatter(acc, [d16 + (2 * j * NP)], fe * w16)
                    plsc.addupdate_scatter(acc, [d16 + ((2 * j + 1) * NP)], fo * w16)
            return c2

        lax.fori_loop(0, CE // (L * GU), grp, 0, unroll=False)

    cp_slab = pltpu.async_copy(vt_hbm.at[wid], slab, sems)
    issue(0, 0)

    def zinit(i, c):
        for j in range(2 * PAIRS):
            acc[pl.ds(j * NP + i * L, L)] = zero
        return c

    lax.fori_loop(0, NP // L, zinit, 0, unroll=False)
    cp_slab.wait()

    def outer(i2, c):
        ia = 2 * i2
        issue(ia + 1, 1)
        wait(0)
        compute(0)

        @pl.when(ia + 2 < NCH4)
        def _():
            issue(ia + 2, 0)

        wait(1)
        compute(1)
        return c

    lax.fori_loop(0, NCH4 // 2, outer, 0, unroll=False)
    pltpu.sync_copy(acc, agg_hbm.at[wid])


def _aggregate(vt_slabs, src, dst, w):
    return pl.kernel(
        _agg_body,
        out_type=jax.ShapeDtypeStruct((NW, 2 * PAIRS * NP), jnp.float32),
        mesh=_sc_mesh(),
        compiler_params=_SC_PARAMS,
        scratch_types=[
            pltpu.VMEM((PAIRS * NP,), jnp.int32),
            pltpu.VMEM((2 * PAIRS * NP,), jnp.float32),
            pltpu.VMEM((CE,), jnp.int32),
            pltpu.VMEM((CE,), jnp.int32),
            pltpu.VMEM((CE,), jnp.float32),
            pltpu.VMEM((CE,), jnp.int32),
            pltpu.VMEM((CE,), jnp.int32),
            pltpu.VMEM((CE,), jnp.float32),
            pltpu.SemaphoreType.DMA,
            pltpu.SemaphoreType.DMA,
            pltpu.SemaphoreType.DMA,
        ],
    )(vt_slabs, src, dst, w)


def _final_body(agg_ref, emb_ref, out_ref):
    out_ref[...] = emb_ref[...] + lax.transpose(agg_ref[...], (1, 0))


def _finalize(agg_t, emb):
    R = 512
    return pl.pallas_call(
        _final_body,
        grid=(NP // R,),
        in_specs=[
            pl.BlockSpec((D, R), lambda i: (0, i)),
            pl.BlockSpec((R, D), lambda i: (i, 0)),
        ],
        out_specs=pl.BlockSpec((R, D), lambda i: (i, 0)),
        out_shape=jax.ShapeDtypeStruct((N_NODES, D), jnp.float32),
    )(agg_t, emb)


def kernel(embeddings, edge_index, Wq, bq, Wk, bk, Wv, bv):
    ws = (Wq[0::2], Wq[1::2], Wk[0::2], Wk[1::2], Wv[0::2], Wv[1::2])
    bs = tuple(b[i::2].reshape(D // 2, 1) for b in (bq, bk, bv) for i in (0, 1))

    src_p, dst_p = _pad_edges(edge_index.astype(jnp.int32))
    src_p = src_p.reshape(NE_PAD)
    dst_p = dst_p.reshape(NE_PAD)

    qtp, ktp, vtp = _project(embeddings, ws, bs)
    sparts = _scores(qtp.reshape(NW, PAIRS * NP), ktp.reshape(NW, PAIRS * NP),
                     dst_p, src_p)
    w = _softmax(sparts).reshape(NE_PAD)

    agg = _aggregate(vtp.reshape(NW, PAIRS * NP), src_p, dst_p, w)
    return _finalize(agg.reshape(D, NP), embeddings)


# speedup vs baseline: 2.5952x; 1.0190x over previous
import jax
import jax.numpy as jnp
from jax import lax
from jax.experimental import pallas as pl
from jax.experimental.pallas import tpu as pltpu
from jax.experimental.pallas import tpu_sc as plsc

N_NODES = 10000
N_EDGES = 160000
D = 256
NC, NS, L = 2, 16, 16
NW = NC * NS
NP = 10240
NE_PAD = 163840

_SC_PARAMS = pltpu.CompilerParams(needs_layout_passes=False)


def _sc_mesh():
    return plsc.VectorSubcoreMesh(core_axis_name="c", subcore_axis_name="s",
                                  num_cores=NC, num_subcores=NS)


def _bf16_bits(x):
    return lax.bitcast_convert_type(x.astype(jnp.bfloat16).astype(jnp.float32),
                                    jnp.int32)


def _pack_pair(even, odd):
    lo = lax.shift_right_logical(_bf16_bits(even), 16)
    hi = jnp.bitwise_and(_bf16_bits(odd), jnp.int32(-65536))
    return jnp.bitwise_or(lo, hi)


EROWS = N_EDGES // 128
EROWS_P = NE_PAD // 128
ERB = EROWS_P // 20


def _proj_body(x_ref, e_ref, wqe_ref, wqo_ref, wke_ref, wko_ref, wve_ref, wvo_ref,
               bqe_ref, bqo_ref, bke_ref, bko_ref, bve_ref, bvo_ref,
               qt_ref, kt_ref, vt_ref, s_ref, d_ref):
    x = x_ref[...]
    inv = 1.0 / (D ** 0.5)
    dn = (((1,), (1,)), ((), ()))

    def mm(w_ref, b_ref):
        return lax.dot_general(w_ref[...], x, dn,
                               preferred_element_type=jnp.float32) + b_ref[...]

    qt_ref[...] = _pack_pair(mm(wqe_ref, bqe_ref) * inv, mm(wqo_ref, bqo_ref) * inv)
    kt_ref[...] = _pack_pair(mm(wke_ref, bke_ref), mm(wko_ref, bko_ref))
    vt_ref[...] = _pack_pair(mm(wve_ref, bve_ref), mm(wvo_ref, bvo_ref))
    i = pl.program_id(0)
    e = e_ref[...]
    valid = i * ERB + lax.broadcasted_iota(jnp.int32, (ERB, 128), 0) < EROWS
    s_ref[...] = jnp.where(valid, e[0], 0)
    d_ref[...] = jnp.where(valid, e[1], 0)


def _project(emb, edge3, ws, bs):
    R = 512
    half = pl.BlockSpec((D // 2, D), lambda i: (0, 0))
    bcol = pl.BlockSpec((D // 2, 1), lambda i: (0, 0))
    outs = pl.BlockSpec((D // 2, R), lambda i: (0, i))
    epad = pl.BlockSpec((ERB, 128), lambda i: (i, 0))
    return pl.pallas_call(
        _proj_body,
        grid=(NP // R,),
        in_specs=[pl.BlockSpec((R, D), lambda i: (i, 0)),
                  pl.BlockSpec((2, ERB, 128), lambda i: (0, i, 0))]
                 + [half] * 6 + [bcol] * 6,
        out_specs=[outs] * 3 + [epad] * 2,
        out_shape=[jax.ShapeDtypeStruct((D // 2, NP), jnp.int32)] * 3
                  + [jax.ShapeDtypeStruct((EROWS_P, 128), jnp.int32)] * 2,
    )(emb, edge3, *ws, *bs)


C2 = 2048
NCH2 = NE_PAD // C2
GU2 = 8


def _scores_body(qt_hbm, kt_hbm, dst_hbm, src_hbm, sp_hbm,
                 qslab, kslab, si0, di0, si1, di1, pb0, pb1,
                 semq, sem0, sem1, semp):
    cid = lax.axis_index("c")
    sid = lax.axis_index("s")
    wid = sid * NC + cid
    ibufs = ((si0, di0, sem0), (si1, di1, sem1))
    pbufs = (pb0, pb1)

    def issue(i, b):
        si, di, sem = ibufs[b]
        off = i * C2
        pltpu.async_copy(src_hbm.at[pl.ds(off, C2)], si, sem)
        pltpu.async_copy(dst_hbm.at[pl.ds(off, C2)], di, sem)

    def wait(b):
        si, di, sem = ibufs[b]
        pltpu.make_async_copy(src_hbm.at[pl.ds(0, C2)], si, sem).wait()
        pltpu.make_async_copy(dst_hbm.at[pl.ds(0, C2)], di, sem).wait()

    def compute(i, b):
        si, di, _ = ibufs[b]
        pbuf = pbufs[b]

        def grp(t, c2):
            for u in range(GU2):
                g = t * GU2 + u
                s16 = si[pl.ds(g * L, L)]
                d16 = di[pl.ds(g * L, L)]
                acc = jnp.zeros((L,), jnp.float32)
                for j in range(PAIRS):
                    qw = plsc.load_gather(qslab, [d16 + (j * NP)])
                    kw = plsc.load_gather(kslab, [s16 + (j * NP)])
                    qe = plsc.bitcast(lax.shift_left(qw, 16), jnp.float32)
                    ke = plsc.bitcast(lax.shift_left(kw, 16), jnp.float32)
                    qo = plsc.bitcast(jnp.bitwise_and(qw, jnp.int32(M_HI)), jnp.float32)
                    ko = plsc.bitcast(jnp.bitwise_and(kw, jnp.int32(M_HI)), jnp.float32)
                    acc = acc + qe * ke + qo * ko
                pbuf[pl.ds(g * L, L)] = acc
            return c2

        lax.fori_loop(0, C2 // (L * GU2), grp, 0, unroll=False)
        pltpu.async_copy(pbuf, sp_hbm.at[wid, pl.ds(i * C2, C2)], semp)

    def wait_pbuf(b):
        pltpu.make_async_copy(pbufs[b], sp_hbm.at[wid, pl.ds(0, C2)], semp).wait()

    cpq = pltpu.async_copy(qt_hbm.at[wid], qslab, semq)
    cpk = pltpu.async_copy(kt_hbm.at[wid], kslab, semq)
    issue(0, 0)
    cpq.wait()
    cpk.wait()

    def outer(i2, c):
        a = 2 * i2
        issue(a + 1, 1)
        wait(0)

        @pl.when(i2 > 0)
        def _():
            wait_pbuf(0)

        compute(a, 0)

        @pl.when(a + 2 < NCH2)
        def _():
            issue(a + 2, 0)

        wait(1)

        @pl.when(i2 > 0)
        def _():
            wait_pbuf(1)

        compute(a + 1, 1)
        return c

    lax.fori_loop(0, NCH2 // 2, outer, 0, unroll=False)
    wait_pbuf(0)
    wait_pbuf(1)


def _scores(qt_slabs, kt_slabs, dst, src):
    return pl.kernel(
        _scores_body,
        out_type=jax.ShapeDtypeStruct((NW, NE_PAD), jnp.float32),
        mesh=_sc_mesh(),
        compiler_params=_SC_PARAMS,
        scratch_types=[
            pltpu.VMEM((PAIRS * NP,), jnp.int32),
            pltpu.VMEM((PAIRS * NP,), jnp.int32),
            pltpu.VMEM((C2,), jnp.int32),
            pltpu.VMEM((C2,), jnp.int32),
            pltpu.VMEM((C2,), jnp.int32),
            pltpu.VMEM((C2,), jnp.int32),
            pltpu.VMEM((C2,), jnp.float32),
            pltpu.VMEM((C2,), jnp.float32),
            pltpu.SemaphoreType.DMA,
            pltpu.SemaphoreType.DMA,
            pltpu.SemaphoreType.DMA,
            pltpu.SemaphoreType.DMA,
        ],
    )(qt_slabs, kt_slabs, dst, src)


SM_ROWS = NE_PAD // 128


def _softmax_body(sp_ref, w_ref):
    s = jnp.sum(sp_ref[...], axis=0)
    rows = lax.broadcasted_iota(jnp.int32, (SM_ROWS, 128), 0)
    cols = lax.broadcasted_iota(jnp.int32, (SM_ROWS, 128), 1)
    valid = rows * 128 + cols < N_EDGES
    s = jnp.where(valid, s, -jnp.inf)
    m = jnp.max(s)
    e = jnp.where(valid, jnp.exp(s - m), 0.0)
    w_ref[...] = e / jnp.sum(e)


def _softmax(sparts):
    return pl.pallas_call(
        _softmax_body,
        out_shape=jax.ShapeDtypeStruct((SM_ROWS, 128), jnp.float32),
    )(sparts.reshape(NW, SM_ROWS, 128))


PAIRS = 4
CE = 800
NCH4 = N_EDGES // CE
GU = 10
M_HI = -65536


def _agg_body(vt_hbm, src_hbm, dst_hbm, w_hbm, agg_hbm,
              slab, acc, si0, di0, wv0, si1, di1, wv1, sem0, sem1, sems):
    cid = lax.axis_index("c")
    sid = lax.axis_index("s")
    wid = sid * NC + cid
    zero = jnp.zeros((L,), jnp.float32)
    bufs = ((si0, di0, wv0, sem0), (si1, di1, wv1, sem1))

    def issue(i, b):
        si, di, wv, sem = bufs[b]
        off = i * CE
        pltpu.async_copy(src_hbm.at[pl.ds(off, CE)], si, sem)
        pltpu.async_copy(dst_hbm.at[pl.ds(off, CE)], di, sem)
        pltpu.async_copy(w_hbm.at[pl.ds(off, CE)], wv, sem)

    def wait(b):
        si, di, wv, sem = bufs[b]
        pltpu.make_async_copy(src_hbm.at[pl.ds(0, CE)], si, sem).wait()
        pltpu.make_async_copy(dst_hbm.at[pl.ds(0, CE)], di, sem).wait()
        pltpu.make_async_copy(w_hbm.at[pl.ds(0, CE)], wv, sem).wait()

    def compute(b):
        si, di, wv, _ = bufs[b]

        def grp(i, c2):
            for u in range(GU):
                g = i * GU + u
                s16 = si[pl.ds(g * L, L)]
                d16 = di[pl.ds(g * L, L)]
                w16 = wv[pl.ds(g * L, L)]
                for j in range(PAIRS):
                    word = plsc.load_gather(slab, [s16 + (j * NP)])
                    fe = plsc.bitcast(lax.shift_left(word, 16), jnp.float32)
                    fo = plsc.bitcast(jnp.bitwise_and(word, jnp.int32(M_HI)), jnp.float32)
                    plsc.addupdate_scatter(acc, [d16 + (2 * j * NP)], fe * w16)
                    plsc.addupdate_scatter(acc, [d16 + ((2 * j + 1) * NP)], fo * w16)
            return c2

        lax.fori_loop(0, CE // (L * GU), grp, 0, unroll=False)

    cp_slab = pltpu.async_copy(vt_hbm.at[wid], slab, sems)
    issue(0, 0)

    def zinit(i, c):
        for j in range(2 * PAIRS):
            acc[pl.ds(j * NP + i * L, L)] = zero
        return c

    lax.fori_loop(0, NP // L, zinit, 0, unroll=False)
    cp_slab.wait()

    def outer(i2, c):
        ia = 2 * i2
        issue(ia + 1, 1)
        wait(0)
        compute(0)

        @pl.when(ia + 2 < NCH4)
        def _():
            issue(ia + 2, 0)

        wait(1)
        compute(1)
        return c

    lax.fori_loop(0, NCH4 // 2, outer, 0, unroll=False)
    for jj in range(2 * PAIRS):
        pltpu.async_copy(acc.at[pl.ds(jj * NP, NP)], agg_hbm.at[8 * wid + jj], sems)
    for jj in range(2 * PAIRS):
        pltpu.make_async_copy(acc.at[pl.ds(jj * NP, NP)], agg_hbm.at[8 * wid + jj],
                              sems).wait()


def _aggregate(vt_slabs, src, dst, w):
    return pl.kernel(
        _agg_body,
        out_type=jax.ShapeDtypeStruct((D, NP), jnp.float32),
        mesh=_sc_mesh(),
        compiler_params=_SC_PARAMS,
        scratch_types=[
            pltpu.VMEM((PAIRS * NP,), jnp.int32),
            pltpu.VMEM((2 * PAIRS * NP,), jnp.float32),
            pltpu.VMEM((CE,), jnp.int32),
            pltpu.VMEM((CE,), jnp.int32),
            pltpu.VMEM((CE,), jnp.float32),
            pltpu.VMEM((CE,), jnp.int32),
            pltpu.VMEM((CE,), jnp.int32),
            pltpu.VMEM((CE,), jnp.float32),
            pltpu.SemaphoreType.DMA,
            pltpu.SemaphoreType.DMA,
            pltpu.SemaphoreType.DMA,
        ],
    )(vt_slabs, src, dst, w)


def _final_body(agg_ref, emb_ref, out_ref):
    out_ref[...] = emb_ref[...] + lax.transpose(agg_ref[...], (1, 0))


def _finalize(agg_t, emb):
    R = 512
    return pl.pallas_call(
        _final_body,
        grid=(NP // R,),
        in_specs=[
            pl.BlockSpec((D, R), lambda i: (0, i)),
            pl.BlockSpec((R, D), lambda i: (i, 0)),
        ],
        out_specs=pl.BlockSpec((R, D), lambda i: (i, 0)),
        out_shape=jax.ShapeDtypeStruct((N_NODES, D), jnp.float32),
    )(agg_t, emb)


def kernel(embeddings, edge_index, Wq, bq, Wk, bk, Wv, bv):
    ws = (Wq[0::2], Wq[1::2], Wk[0::2], Wk[1::2], Wv[0::2], Wv[1::2])
    bs = tuple(b[i::2].reshape(D // 2, 1) for b in (bq, bk, bv) for i in (0, 1))

    edge3 = edge_index.astype(jnp.int32).reshape(2, EROWS, 128)
    qtp, ktp, vtp, src_p, dst_p = _project(embeddings, edge3, ws, bs)
    src_p = src_p.reshape(NE_PAD)
    dst_p = dst_p.reshape(NE_PAD)
    sparts = _scores(qtp.reshape(NW, PAIRS * NP), ktp.reshape(NW, PAIRS * NP),
                     dst_p, src_p)
    w = _softmax(sparts).reshape(NE_PAD)

    agg = _aggregate(vtp.reshape(NW, PAIRS * NP), src_p, dst_p, w)
    return _finalize(agg, embeddings)
